# Initial kernel scaffold; baseline (speedup 1.0000x reference)
import jax
import jax.numpy as jnp
from jax.experimental import pallas as pl


def kernel(queries, keys, values, Wq, bq, Wk, bk, Wv, bv, Wo, bo):
    raise NotImplementedError("write your pallas kernel here")


# speedup vs baseline: 5.0794x; 5.0794x over previous
import math

import jax
import jax.numpy as jnp
import numpy as np
from jax.experimental import pallas as pl
from jax.experimental.pallas import tpu as pltpu

_B, _S, _D, _H = 2, 2048, 768, 12
_DH = _D // _H
_BH = _B * _H
_RAND = 5 * int(np.ceil(np.log(_S)))
_TOP = 5 * int(np.log(_S))
_SCALE = 1.0 / math.sqrt(_DH)
_BLK = 256
_NBLK = _S // _BLK

_rand_idx = np.asarray(jax.random.randint(jax.random.key(42), (_S, _RAND), 0, _S))
_cnt_T = np.zeros((_S, _S), np.int8)
np.add.at(_cnt_T, (_rand_idx.ravel(), np.repeat(np.arange(_S), _RAND)), 1)
_LTRI = np.tril(np.ones((_BLK, _BLK), np.float32))


def _linear_kern(x_ref, w_ref, b_ref, o_ref):
    o_ref[0, 0] = (
        jnp.dot(x_ref[0, 0], w_ref[0], preferred_element_type=jnp.float32)
        + b_ref[0]
    )


def _linear(xs, ws, bs, n_stack, sb=512):
    return pl.pallas_call(
        _linear_kern,
        grid=(n_stack, _B, _S // sb),
        in_specs=[
            pl.BlockSpec((1, 1, sb, _D), lambda i, b, s: (i, b, s, 0)),
            pl.BlockSpec((1, _D, _D), lambda i, b, s: (i, 0, 0)),
            pl.BlockSpec((1, 1, _D), lambda i, b, s: (i, 0, 0)),
        ],
        out_specs=pl.BlockSpec((1, 1, sb, _D), lambda i, b, s: (i, b, s, 0)),
        out_shape=jax.ShapeDtypeStruct((n_stack, _B, _S, _D), jnp.float32),
    )(xs, ws, bs)


def _attn_kern(qT_ref, kT_ref, q_ref, k_ref, v_ref, cnt_ref, ltri_ref,
               o_ref, qi_ref, idx_ref):
    def blk_body(r, carry):
        smax, ssum = carry
        kb = k_ref[0, pl.ds(r * _BLK, _BLK), :]
        at = jnp.dot(kb, qT_ref[0], preferred_element_type=jnp.float32)
        cf = cnt_ref[pl.ds(r * _BLK, _BLK), :].astype(jnp.float32)
        masked = jnp.where(cf > 0.0, at, -1e30)
        smax = jnp.maximum(smax, jnp.max(masked, axis=0, keepdims=True))
        ssum = ssum + jnp.sum(at * cf, axis=0, keepdims=True)
        return smax, ssum

    init = (jnp.full((1, _S), -1e30, jnp.float32), jnp.zeros((1, _S), jnp.float32))
    smax, ssum = jax.lax.fori_loop(0, _NBLK, blk_body, init)
    disc = smax - ssum / _S

    lane = jax.lax.broadcasted_iota(jnp.int32, (1, _S), 1)
    qi_ref[...] = jnp.zeros((_RAND, _DH), jnp.float32)

    def top_body(i, dcur):
        m = jnp.max(dcur)
        idx = jnp.min(jnp.where(dcur == m, lane, _S))
        idx_ref[i] = idx
        qi_ref[pl.ds(i, 1), :] = q_ref[0, pl.ds(idx, 1), :]
        return jnp.where(lane == idx, -3e38, dcur)

    jax.lax.fori_loop(0, _TOP, top_body, disc)

    qk = jnp.dot(qi_ref[...], kT_ref[0], preferred_element_type=jnp.float32) * _SCALE
    m = jnp.max(qk, axis=1, keepdims=True)
    e = jnp.exp(qk - m)
    p = e / jnp.sum(e, axis=1, keepdims=True)
    upd = jnp.dot(p, v_ref[0], preferred_element_type=jnp.float32)

    def cum_body(r, carry):
        vb = v_ref[0, pl.ds(r * _BLK, _BLK), :]
        o_ref[0, pl.ds(r * _BLK, _BLK), :] = (
            jnp.dot(ltri_ref[...], vb, preferred_element_type=jnp.float32) + carry
        )
        return carry + jnp.sum(vb, axis=0, keepdims=True)

    jax.lax.fori_loop(0, _NBLK, cum_body, jnp.zeros((1, _DH), jnp.float32))

    qi_ref[...] = upd

    def scat_body(i, c):
        s = idx_ref[i]
        o_ref[0, pl.ds(s, 1), :] = qi_ref[pl.ds(i, 1), :]
        return c

    jax.lax.fori_loop(0, _TOP, scat_body, 0)


def _attn(qT, kT, q3, k3, v3, cntT, ltri):
    return pl.pallas_call(
        _attn_kern,
        grid=(_BH,),
        in_specs=[
            pl.BlockSpec((1, _DH, _S), lambda g: (g, 0, 0)),
            pl.BlockSpec((1, _DH, _S), lambda g: (g, 0, 0)),
            pl.BlockSpec((1, _S, _DH), lambda g: (g, 0, 0)),
            pl.BlockSpec((1, _S, _DH), lambda g: (g, 0, 0)),
            pl.BlockSpec((1, _S, _DH), lambda g: (g, 0, 0)),
            pl.BlockSpec((_S, _S), lambda g: (0, 0)),
            pl.BlockSpec((_BLK, _BLK), lambda g: (0, 0)),
        ],
        out_specs=pl.BlockSpec((1, _S, _DH), lambda g: (g, 0, 0)),
        out_shape=jax.ShapeDtypeStruct((_BH, _S, _DH), jnp.float32),
        scratch_shapes=[
            pltpu.VMEM((_RAND, _DH), jnp.float32),
            pltpu.SMEM((_RAND,), jnp.int32),
        ],
    )(qT, kT, q3, k3, v3, cntT, ltri)


def kernel(queries, keys, values, Wq, bq, Wk, bk, Wv, bv, Wo, bo):
    xs = jnp.stack([queries, keys, values])
    ws = jnp.stack([Wq.T, Wk.T, Wv.T])
    bs = jnp.stack([bq, bk, bv])[:, None, :]
    qkv = _linear(xs, ws, bs, 3)

    qkvh = (
        qkv.reshape(3, _B, _S, _H, _DH)
        .transpose(0, 1, 3, 2, 4)
        .reshape(3, _BH, _S, _DH)
    )
    q3, k3, v3 = qkvh[0], qkvh[1], qkvh[2]
    qT = q3.transpose(0, 2, 1)
    kT = k3.transpose(0, 2, 1)

    vc = _attn(qT, kT, q3, k3, v3, jnp.asarray(_cnt_T), jnp.asarray(_LTRI))

    vc2 = (
        vc.reshape(_B, _H, _S, _DH)
        .transpose(0, 2, 1, 3)
        .reshape(1, _B, _S, _D)
    )
    out = _linear(vc2, Wo.T[None], bo[None, None, :], 1)
    return out[0]


# speedup vs baseline: 5.5091x; 1.0846x over previous
import math

import jax
import jax.numpy as jnp
import numpy as np
from jax.experimental import pallas as pl
from jax.experimental.pallas import tpu as pltpu

_B, _S, _D, _H = 2, 2048, 768, 12
_DH = _D // _H
_BH = _B * _H
_RAND = 5 * int(np.ceil(np.log(_S)))
_TOP = 5 * int(np.log(_S))
_SCALE = 1.0 / math.sqrt(_DH)
_BLK = 256
_NBLK = _S // _BLK

def _threefry_pair(keypair, x0, x1):
    rot1 = (13, 15, 26, 6)
    rot2 = (17, 29, 16, 24)

    def rotl(x, r):
        return (x << np.uint32(r)) | (x >> np.uint32(32 - r))

    x0 = x0.astype(np.uint32).copy()
    x1 = x1.astype(np.uint32).copy()
    ks0, ks1 = np.uint32(keypair[0]), np.uint32(keypair[1])
    ks2 = ks0 ^ ks1 ^ np.uint32(0x1BD11BDA)
    sched = [(rot1, ks1, ks2), (rot2, ks2, ks0), (rot1, ks0, ks1),
             (rot2, ks1, ks2), (rot1, ks2, ks0)]
    with np.errstate(over="ignore"):
        x0 = x0 + ks0
        x1 = x1 + ks1
        for i, (rots, a0, a1) in enumerate(sched):
            for r in rots:
                x0 = x0 + x1
                x1 = rotl(x1, r) ^ x0
            x0 = x0 + a0
            x1 = x1 + a1 + np.uint32(i + 1)
    return x0, x1


def _rand_index():
    root = (np.uint32(0), np.uint32(42))
    z = np.zeros(2, np.uint32)
    b1, b2 = _threefry_pair(root, z, np.arange(2, dtype=np.uint32))
    child = (b1[1], b2[1])
    n = _S * _RAND
    o1, o2 = _threefry_pair(child, np.zeros(n, np.uint32),
                            np.arange(n, dtype=np.uint32))
    bits = o1 ^ o2
    return (bits % np.uint32(_S)).astype(np.int32).reshape(_S, _RAND)


_CONSTS: list = []


def _consts():
    if not _CONSTS:
        ri = _rand_index()
        cnt_T = np.zeros((_S, _S), np.int8)
        np.add.at(cnt_T, (ri.ravel(), np.repeat(np.arange(_S), _RAND)), 1)
        ltri = np.tril(np.ones((_BLK, _BLK), np.float32))
        _CONSTS.append((cnt_T, ltri))
    return _CONSTS[0]


def _linear_kern(x_ref, w_ref, b_ref, o_ref):
    o_ref[0, 0] = (
        jnp.dot(x_ref[0, 0], w_ref[0], preferred_element_type=jnp.float32)
        + b_ref[0]
    )


def _linear(xs, ws, bs, n_stack, sb=512):
    return pl.pallas_call(
        _linear_kern,
        grid=(n_stack, _B, _S // sb),
        in_specs=[
            pl.BlockSpec((1, 1, sb, _D), lambda i, b, s: (i, b, s, 0)),
            pl.BlockSpec((1, _D, _D), lambda i, b, s: (i, 0, 0)),
            pl.BlockSpec((1, 1, _D), lambda i, b, s: (i, 0, 0)),
        ],
        out_specs=pl.BlockSpec((1, 1, sb, _D), lambda i, b, s: (i, b, s, 0)),
        out_shape=jax.ShapeDtypeStruct((n_stack, _B, _S, _D), jnp.float32),
    )(xs, ws, bs)


_NT = (((1,), (1,)), ((), ()))


def _attn_kern(q_ref, k_ref, v_ref, cnt_ref, ltri_ref, o_ref, qi_ref, idx_ref):
    def blk_body(r, carry):
        smax, ssum = carry
        kb = k_ref[0, 0, pl.ds(r * _BLK, _BLK), :]
        at = jax.lax.dot_general(kb, q_ref[0, 0], _NT,
                                 preferred_element_type=jnp.float32)
        cf = cnt_ref[pl.ds(r * _BLK, _BLK), :].astype(jnp.float32)
        masked = jnp.where(cf > 0.0, at, -1e30)
        smax = jnp.maximum(smax, jnp.max(masked, axis=0, keepdims=True))
        ssum = ssum + jnp.sum(at * cf, axis=0, keepdims=True)
        return smax, ssum

    init = (jnp.full((1, _S), -1e30, jnp.float32), jnp.zeros((1, _S), jnp.float32))
    smax, ssum = jax.lax.fori_loop(0, _NBLK, blk_body, init)
    disc = smax - ssum / _S

    lane = jax.lax.broadcasted_iota(jnp.int32, (1, _S), 1)
    qi_ref[...] = jnp.zeros((_RAND, _DH), jnp.float32)

    def top_body(i, dcur):
        m = jnp.max(dcur)
        idx = jnp.min(jnp.where(dcur == m, lane, _S))
        idx_ref[i] = idx
        qi_ref[pl.ds(i, 1), :] = q_ref[0, 0, pl.ds(idx, 1), :]
        return jnp.where(lane == idx, -3e38, dcur)

    jax.lax.fori_loop(0, _TOP, top_body, disc)

    qk = jax.lax.dot_general(qi_ref[...], k_ref[0, 0], _NT,
                             preferred_element_type=jnp.float32) * _SCALE
    m = jnp.max(qk, axis=1, keepdims=True)
    e = jnp.exp(qk - m)
    p = e / jnp.sum(e, axis=1, keepdims=True)
    upd = jnp.dot(p, v_ref[0, 0], preferred_element_type=jnp.float32)

    def cum_body(r, carry):
        vb = v_ref[0, 0, pl.ds(r * _BLK, _BLK), :]
        o_ref[0, pl.ds(r * _BLK, _BLK), :] = (
            jnp.dot(ltri_ref[...], vb, preferred_element_type=jnp.float32) + carry
        )
        return carry + jnp.sum(vb, axis=0, keepdims=True)

    jax.lax.fori_loop(0, _NBLK, cum_body, jnp.zeros((1, _DH), jnp.float32))

    qi_ref[...] = upd

    def scat_body(i, c):
        s = idx_ref[i]
        o_ref[0, pl.ds(s, 1), :] = qi_ref[pl.ds(i, 1), :]
        return c

    jax.lax.fori_loop(0, _TOP, scat_body, 0)


def _attn(qkvh, cntT, ltri):
    head_spec = lambda i: pl.BlockSpec(
        (1, 1, _S, _DH), lambda g, i=i: (i, g, 0, 0))
    return pl.pallas_call(
        _attn_kern,
        grid=(_BH,),
        in_specs=[
            head_spec(0),
            head_spec(1),
            head_spec(2),
            pl.BlockSpec((_S, _S), lambda g: (0, 0)),
            pl.BlockSpec((_BLK, _BLK), lambda g: (0, 0)),
        ],
        out_specs=pl.BlockSpec((1, _S, _DH), lambda g: (g, 0, 0)),
        out_shape=jax.ShapeDtypeStruct((_BH, _S, _DH), jnp.float32),
        scratch_shapes=[
            pltpu.VMEM((_RAND, _DH), jnp.float32),
            pltpu.SMEM((_RAND,), jnp.int32),
        ],
    )(qkvh, qkvh, qkvh, cntT, ltri)


def kernel(queries, keys, values, Wq, bq, Wk, bk, Wv, bv, Wo, bo):
    xs = jnp.stack([queries, keys, values])
    ws = jnp.stack([Wq.T, Wk.T, Wv.T])
    bs = jnp.stack([bq, bk, bv])[:, None, :]
    qkv = _linear(xs, ws, bs, 3)

    qkvh = (
        qkv.reshape(3, _B, _S, _H, _DH)
        .transpose(0, 1, 3, 2, 4)
        .reshape(3, _BH, _S, _DH)
    )
    cnt_T, ltri = _consts()
    vc = _attn(qkvh, jnp.asarray(cnt_T), jnp.asarray(ltri))

    vc2 = (
        vc.reshape(_B, _H, _S, _DH)
        .transpose(0, 2, 1, 3)
        .reshape(1, _B, _S, _D)
    )
    out = _linear(vc2, Wo.T[None], bo[None, None, :], 1)
    return out[0]
